# baseline (speedup 1.0000x reference)
import functools

import jax
import jax.numpy as jnp
from jax import lax
from jax.experimental import pallas as pl
from jax.experimental.pallas import tpu as pltpu
from jax.experimental.pallas import tpu_sc as plsc

E = 8
K = 2
H = 2048
T = 8192
BT = 2048

NC = 1
NS = 16
L = 16
NW = NC * NS
TPW = T // NW
NCHUNK = TPW // L
NEG_INF = float("-inf")


def _matmul_body(x_ref, w_ref, lg_ref):
    lg_ref[...] = jax.lax.dot_general(
        w_ref[...], x_ref[...], (((1,), (1,)), ((), ())),
        preferred_element_type=jnp.float32,
    )


def _logits_t(input, weight):
    return pl.pallas_call(
        _matmul_body,
        grid=(T // BT,),
        in_specs=[
            pl.BlockSpec((BT, H), lambda t: (t, 0)),
            pl.BlockSpec((E, H), lambda t: (0, 0)),
        ],
        out_specs=pl.BlockSpec((E, BT), lambda t: (0, t)),
        out_shape=jax.ShapeDtypeStruct((E, T), jnp.float32),
        compiler_params=pltpu.CompilerParams(
            dimension_semantics=("arbitrary",),
        ),
    )(input, weight)


_sc_mesh = plsc.VectorSubcoreMesh(
    core_axis_name="c", subcore_axis_name="s", num_cores=NC
)

_GATHER_DN = lax.GatherDimensionNumbers(
    offset_dims=(), collapsed_slice_dims=(0,), start_index_map=(0,)
)


def _shuffle(v, idx):
    return lax.gather(
        v, idx[:, None], _GATHER_DN, (1,),
        mode=lax.GatherScatterMode.PROMISE_IN_BOUNDS,
    )


@functools.partial(
    pl.kernel,
    out_type=[
        jax.ShapeDtypeStruct((K, T), jnp.float32),
        jax.ShapeDtypeStruct((K, T), jnp.int32),
        jax.ShapeDtypeStruct((NW, L), jnp.float32),
    ],
    mesh=_sc_mesh,
    scratch_types=[
        pltpu.VMEM((E, TPW), jnp.float32),
        pltpu.VMEM((K, TPW), jnp.float32),
        pltpu.VMEM((K, TPW), jnp.int32),
        pltpu.VMEM((L,), jnp.float32),
    ],
)
def _route_sc(lg_hbm, sc_hbm, ix_hbm, cnt_hbm, lg_v, sc_v, ix_v, cnt_v):
    cid = lax.axis_index("c")
    sid = lax.axis_index("s")
    wid = sid * NC + cid
    base = wid * TPW

    pltpu.sync_copy(lg_hbm.at[:, pl.ds(base, TPW)], lg_v)

    acc = [jnp.zeros((L,), jnp.float32) for _ in range(E)]
    one = jnp.ones((L,), jnp.float32)
    zero = jnp.zeros((L,), jnp.float32)

    for i in range(NCHUNK):
        sl = pl.ds(i * L, L)
        ls = [lg_v[e, sl] for e in range(E)]
        m1 = ls[0]
        i1 = jnp.zeros((L,), jnp.int32)
        for e in range(1, E):
            take = ls[e] > m1
            m1 = jnp.where(take, ls[e], m1)
            i1 = jnp.where(take, e, i1)
        m2 = jnp.full((L,), NEG_INF, jnp.float32)
        i2 = jnp.zeros((L,), jnp.int32)
        for e in range(E):
            le = jnp.where(i1 == e, NEG_INF, ls[e])
            take = le > m2
            m2 = jnp.where(take, le, m2)
            i2 = jnp.where(take, e, i2)
        d = jnp.exp(m2 - m1)
        s1 = 1.0 / (1.0 + d)
        sc_v[0, sl] = s1
        sc_v[1, sl] = d * s1
        ix_v[0, sl] = i1
        ix_v[1, sl] = i2
        for e in range(E):
            acc[e] = (acc[e]
                      + jnp.where(i1 == e, one, zero)
                      + jnp.where(i2 == e, one, zero))

    lane = lax.iota(jnp.int32, L)
    cnt = jnp.zeros((L,), jnp.float32)
    for e in range(E):
        tot = acc[e]
        for sh in (8, 4, 2, 1):
            tot = tot + _shuffle(tot, lane ^ sh)
        cnt = jnp.where(lane == e, tot, cnt)
    cnt_v[...] = cnt

    pltpu.sync_copy(sc_v, sc_hbm.at[:, pl.ds(base, TPW)])
    pltpu.sync_copy(ix_v, ix_hbm.at[:, pl.ds(base, TPW)])
    pltpu.sync_copy(cnt_v, cnt_hbm.at[wid])


@jax.jit
def kernel(input, weight):
    logits_t = _logits_t(input, weight)
    scores_t, idx_t, cnt = _route_sc(logits_t)
    return scores_t.T, idx_t.T, cnt.sum(axis=0)[:E]


# speedup vs baseline: 1.0705x; 1.0705x over previous
import functools

import jax
import jax.numpy as jnp
from jax import lax
from jax.experimental import pallas as pl
from jax.experimental.pallas import tpu as pltpu
from jax.experimental.pallas import tpu_sc as plsc

E = 8
K = 2
H = 2048
T = 8192
BT = 2048

NC = 1
NS = 16
L = 16
NW = NC * NS
TPW = T // NW
NCHUNK = TPW // L
NEG_INF = float("-inf")


def _matmul_hist_body(x_ref, w_ref, lg_ref, cnt_ref):
    x = x_ref[...]
    w = w_ref[...]
    lg = jax.lax.dot_general(
        w, x, (((1,), (1,)), ((), ())), preferred_element_type=jnp.float32
    )
    lg_ref[...] = lg

    eidx = jax.lax.broadcasted_iota(jnp.int32, (E, BT), 0)
    m1 = jnp.max(lg, axis=0, keepdims=True)
    i1 = jnp.min(jnp.where(lg == m1, eidx, E), axis=0, keepdims=True)
    masked = jnp.where(eidx == i1, NEG_INF, lg)
    m2 = jnp.max(masked, axis=0, keepdims=True)
    i2 = jnp.min(jnp.where(masked == m2, eidx, E), axis=0, keepdims=True)
    onehot = (eidx == i1).astype(jnp.float32) + (eidx == i2).astype(jnp.float32)
    part = jnp.sum(onehot, axis=1, keepdims=True)

    @pl.when(pl.program_id(0) == 0)
    def _init():
        cnt_ref[...] = jnp.zeros_like(cnt_ref)

    cnt_ref[...] += part


def _logits_t_and_counts(input, weight):
    return pl.pallas_call(
        _matmul_hist_body,
        grid=(T // BT,),
        in_specs=[
            pl.BlockSpec((BT, H), lambda t: (t, 0)),
            pl.BlockSpec((E, H), lambda t: (0, 0)),
        ],
        out_specs=[
            pl.BlockSpec((E, BT), lambda t: (0, t)),
            pl.BlockSpec((E, 1), lambda t: (0, 0)),
        ],
        out_shape=[
            jax.ShapeDtypeStruct((E, T), jnp.float32),
            jax.ShapeDtypeStruct((E, 1), jnp.float32),
        ],
        compiler_params=pltpu.CompilerParams(
            dimension_semantics=("arbitrary",),
        ),
    )(input, weight)


_sc_mesh = plsc.VectorSubcoreMesh(core_axis_name="c", subcore_axis_name="s", num_cores=1)


@functools.partial(
    pl.kernel,
    out_type=[
        jax.ShapeDtypeStruct((K, T), jnp.float32),
        jax.ShapeDtypeStruct((K, T), jnp.int32),
    ],
    mesh=_sc_mesh,
    scratch_types=[
        pltpu.VMEM((E, TPW), jnp.float32),
        pltpu.VMEM((K, TPW), jnp.float32),
        pltpu.VMEM((K, TPW), jnp.int32),
    ],
)
def _route_sc(lg_hbm, sc_hbm, ix_hbm, lg_v, sc_v, ix_v):
    cid = lax.axis_index("c")
    sid = lax.axis_index("s")
    wid = sid * NC + cid
    base = wid * TPW

    pltpu.sync_copy(lg_hbm.at[:, pl.ds(base, TPW)], lg_v)

    for i in range(NCHUNK):
        sl = pl.ds(i * L, L)
        ls = [lg_v[e, sl] for e in range(E)]
        m1 = ls[0]
        i1 = jnp.zeros((L,), jnp.int32)
        for e in range(1, E):
            take = ls[e] > m1
            m1 = jnp.where(take, ls[e], m1)
            i1 = jnp.where(take, e, i1)
        m2 = jnp.full((L,), NEG_INF, jnp.float32)
        i2 = jnp.zeros((L,), jnp.int32)
        for e in range(E):
            le = jnp.where(i1 == e, NEG_INF, ls[e])
            take = le > m2
            m2 = jnp.where(take, le, m2)
            i2 = jnp.where(take, e, i2)
        d = jnp.exp(m2 - m1)
        s1 = 1.0 / (1.0 + d)
        sc_v[0, sl] = s1
        sc_v[1, sl] = d * s1
        ix_v[0, sl] = i1
        ix_v[1, sl] = i2

    pltpu.sync_copy(sc_v, sc_hbm.at[:, pl.ds(base, TPW)])
    pltpu.sync_copy(ix_v, ix_hbm.at[:, pl.ds(base, TPW)])


@jax.jit
def kernel(input, weight):
    logits_t, cnt = _logits_t_and_counts(input, weight)
    scores_t, idx_t = _route_sc(logits_t)
    return scores_t.T, idx_t.T, cnt.reshape(E)
